# baseline (speedup 1.0000x reference)
import functools

import jax
import jax.numpy as jnp
from jax import lax
from jax.experimental import pallas as pl
from jax.experimental.pallas import tpu as pltpu
from jax.experimental.pallas import tpu_sc as plsc

COULOMB_CONSTANT = 14.399645478425668
CUTOFF = 10.0
R_ON = 0.8 * CUTOFF
HALF_PI = 1.5707963267948966
PI = 3.141592653589793
SCALE = PI / (CUTOFF - R_ON)

C0 = 0.25 * COULOMB_CONSTANT
C3 = C0 * (-1.0 / 6.0)
C5 = C0 * (1.0 / 120.0)
C7 = C0 * (-1.0 / 5040.0)

NC = 2
NS = 16
NW = NC * NS
L = 16
CT = 512
CH = 5 * CT


def _sc_coulomb(n, n_pad, e_pad):
    total_ch = e_pad // CH
    ch_q, ch_r = divmod(total_ch, NW)

    mesh = plsc.VectorSubcoreMesh(core_axis_name="c", subcore_axis_name="s")

    @functools.partial(
        pl.kernel,
        out_type=jax.ShapeDtypeStruct((NW * n_pad,), jnp.float32),
        mesh=mesh,
        compiler_params=pltpu.CompilerParams(needs_layout_passes=False),
        scratch_types=[
            pltpu.VMEM((n_pad,), jnp.float32),
            pltpu.VMEM((n_pad,), jnp.float32),
            pltpu.VMEM((2, 2 * CH), jnp.int32),
            pltpu.VMEM((2 * CH,), jnp.float32),
            pltpu.SemaphoreType.DMA,
            pltpu.SemaphoreType.DMA,
            pltpu.SemaphoreType.DMA,
        ],
    )
    def kern(idx2_hbm, length_hbm, charges_hbm, out_hbm,
             table, acc, ibuf, lbuf, tsem, sem0, sem1):
        c = lax.axis_index("c")
        s = lax.axis_index("s")
        wid = c * NS + s
        nch_w = ch_q + jnp.where(wid < ch_r, 1, 0)
        start_w = wid * ch_q + jnp.minimum(wid, ch_r)

        def issue(j, half, sem):
            base = (start_w + j) * CH
            off = half * CH
            pltpu.async_copy(idx2_hbm.at[:, pl.ds(base, CH)],
                             ibuf.at[:, pl.ds(off, CH)], sem)
            pltpu.async_copy(length_hbm.at[pl.ds(base, CH)],
                             lbuf.at[pl.ds(off, CH)], sem)

        def drain(half, sem):
            off = half * CH
            pltpu.make_async_copy(idx2_hbm.at[:, pl.ds(0, CH)],
                                  ibuf.at[:, pl.ds(off, CH)], sem).wait()
            pltpu.make_async_copy(length_hbm.at[pl.ds(0, CH)],
                                  lbuf.at[pl.ds(off, CH)], sem).wait()

        def compute(half):
            off = half * CH

            @plsc.parallel_loop(0, CH, L, unroll=4)
            def _(i):
                cidx = ibuf[0, pl.ds(off + i, L)]
                nidx = ibuf[1, pl.ds(off + i, L)]
                d = lbuf[pl.ds(off + i, L)]
                q1 = plsc.load_gather(table, [cidx])
                q2 = plsc.load_gather(table, [nidx])
                x = jnp.clip((d - R_ON) * SCALE, 0.0, PI) - HALF_PI
                x2 = x * x
                sinx_c = x * (C0 + x2 * (C3 + x2 * (C5 + x2 * C7)))
                pair = (C0 - sinx_c) * q1 * q2 / d
                plsc.addupdate_scatter(acc, [cidx], pair)

        tcopy = pltpu.async_copy(charges_hbm, table.at[pl.ds(0, n)], tsem)
        issue(0, 0, sem0)

        zero16 = jnp.zeros((L,), jnp.float32)

        @plsc.parallel_loop(0, n_pad, L, unroll=8)
        def _(i):
            acc[pl.ds(i, L)] = zero16

        tcopy.wait()

        @pl.loop(0, nch_w, step=2)
        def _(j):
            @pl.when(j + 1 < nch_w)
            def _():
                issue(j + 1, 1, sem1)

            drain(0, sem0)
            compute(0)

            @pl.when(j + 2 < nch_w)
            def _():
                issue(j + 2, 0, sem0)

            @pl.when(j + 1 < nch_w)
            def _():
                drain(1, sem1)
                compute(1)

        pltpu.sync_copy(acc, out_hbm.at[pl.ds(wid * n_pad, n_pad)])

    return kern


def _tc_sum(partials_flat, n, n_pad):
    def body(p_hbm, o_ref, buf, sem0, sem1):
        sems = (sem0, sem1)

        def copy(w):
            return pltpu.make_async_copy(
                p_hbm.at[pl.ds(w * n_pad, n_pad)], buf.at[w % 2], sems[w % 2])

        copy(0).start()
        for w in range(NW):
            if w + 1 < NW:
                copy(w + 1).start()
            copy(w).wait()
            if w == 0:
                acc = buf[w % 2]
            else:
                acc = acc + buf[w % 2]
        o_ref[...] = acc[:n]

    return pl.pallas_call(
        body,
        in_specs=[pl.BlockSpec(memory_space=pltpu.MemorySpace.HBM)],
        out_shape=jax.ShapeDtypeStruct((n,), jnp.float32),
        scratch_shapes=[
            pltpu.VMEM((2, n_pad), jnp.float32),
            pltpu.SemaphoreType.DMA,
            pltpu.SemaphoreType.DMA,
        ],
    )(partials_flat)


def kernel(long_edge_index, long_edge_length, atomic_charges):
    n = atomic_charges.shape[0]
    e = long_edge_length.shape[0]

    n_pad = ((n + 1 + 255) // 256) * 256
    e_pad = ((e + CH - 1) // CH) * CH

    length = long_edge_length.astype(jnp.float32)
    idx2 = long_edge_index.astype(jnp.int32)
    charges = atomic_charges.astype(jnp.float32)
    if e_pad != e:
        idx2 = jnp.pad(idx2, ((0, 0), (0, e_pad - e)), constant_values=n)
        length = jnp.pad(length, (0, e_pad - e), constant_values=1.0)
        charges = jnp.pad(charges, (0, n_pad - n))
    partials = _sc_coulomb(charges.shape[0], n_pad, e_pad)(
        idx2, length, charges)
    return _tc_sum(partials, n, n_pad)


# speedup vs baseline: 1.2599x; 1.2599x over previous
import functools

import jax
import jax.numpy as jnp
from jax import lax
from jax.experimental import pallas as pl
from jax.experimental.pallas import tpu as pltpu
from jax.experimental.pallas import tpu_sc as plsc

COULOMB_CONSTANT = 14.399645478425668
CUTOFF = 10.0
R_ON = 0.8 * CUTOFF
HALF_PI = 1.5707963267948966
PI = 3.141592653589793
SCALE = PI / (CUTOFF - R_ON)

C0 = 0.25 * COULOMB_CONSTANT
C3 = C0 * (-1.0 / 6.0)
C5 = C0 * (1.0 / 120.0)
C7 = C0 * (-1.0 / 5040.0)

NC = 2
NS = 16
NW = NC * NS
L = 16
CT = 512
CH = 5 * CT


def _sc_coulomb(n, n_pad, e_pad):
    total_ch = e_pad // CH
    ch_q, ch_r = divmod(total_ch, NW)

    mesh = plsc.VectorSubcoreMesh(core_axis_name="c", subcore_axis_name="s")

    @functools.partial(
        pl.kernel,
        out_type=jax.ShapeDtypeStruct((NW * n_pad,), jnp.float32),
        mesh=mesh,
        compiler_params=pltpu.CompilerParams(needs_layout_passes=False),
        scratch_types=[
            pltpu.VMEM((n_pad,), jnp.float32),
            pltpu.VMEM((n_pad,), jnp.float32),
            pltpu.VMEM((2, 2 * CH), jnp.int32),
            pltpu.VMEM((2 * CH,), jnp.float32),
            pltpu.SemaphoreType.DMA,
            pltpu.SemaphoreType.DMA,
            pltpu.SemaphoreType.DMA,
        ],
    )
    def kern(idx2_hbm, length_hbm, charges_hbm, out_hbm,
             table, acc, ibuf, lbuf, tsem, sem0, sem1):
        c = lax.axis_index("c")
        s = lax.axis_index("s")
        wid = c * NS + s
        nch_w = ch_q + jnp.where(wid < ch_r, 1, 0)
        start_w = wid * ch_q + jnp.minimum(wid, ch_r)

        def issue(j, half, sem):
            base = (start_w + j) * CH
            off = half * CH
            pltpu.async_copy(idx2_hbm.at[:, pl.ds(base, CH)],
                             ibuf.at[:, pl.ds(off, CH)], sem)
            pltpu.async_copy(length_hbm.at[pl.ds(base, CH)],
                             lbuf.at[pl.ds(off, CH)], sem)

        def drain(half, sem):
            off = half * CH
            pltpu.make_async_copy(idx2_hbm.at[:, pl.ds(0, CH)],
                                  ibuf.at[:, pl.ds(off, CH)], sem).wait()
            pltpu.make_async_copy(length_hbm.at[pl.ds(0, CH)],
                                  lbuf.at[pl.ds(off, CH)], sem).wait()

        def compute(half):
            off = half * CH

            @plsc.parallel_loop(0, CH, L, unroll=8)
            def _(i):
                cidx = ibuf[0, pl.ds(off + i, L)]
                nidx = ibuf[1, pl.ds(off + i, L)]
                d = lbuf[pl.ds(off + i, L)]
                q1 = plsc.load_gather(table, [cidx])
                q2 = plsc.load_gather(table, [nidx])
                x = jnp.clip((d - R_ON) * SCALE, 0.0, PI) - HALF_PI
                x2 = x * x
                sinx_c = x * (C0 + x2 * (C3 + x2 * (C5 + x2 * C7)))
                pair = (C0 - sinx_c) * q1 * q2 / d
                plsc.addupdate_scatter(acc, [cidx], pair)

        tcopy = pltpu.async_copy(charges_hbm, table.at[pl.ds(0, n)], tsem)
        issue(0, 0, sem0)

        zero16 = jnp.zeros((L,), jnp.float32)

        @plsc.parallel_loop(0, n_pad, L, unroll=8)
        def _(i):
            acc[pl.ds(i, L)] = zero16

        tcopy.wait()

        @pl.loop(0, nch_w, step=2)
        def _(j):
            @pl.when(j + 1 < nch_w)
            def _():
                issue(j + 1, 1, sem1)

            drain(0, sem0)
            compute(0)

            @pl.when(j + 2 < nch_w)
            def _():
                issue(j + 2, 0, sem0)

            @pl.when(j + 1 < nch_w)
            def _():
                drain(1, sem1)
                compute(1)

        pltpu.sync_copy(acc, out_hbm.at[pl.ds(wid * n_pad, n_pad)])

    return kern


def _tc_sum(partials_flat, n, n_pad):
    def body(p_ref, o_ref):
        acc = p_ref[pl.ds(0, n_pad)]
        for w in range(1, NW):
            acc = acc + p_ref[pl.ds(w * n_pad, n_pad)]
        o_ref[...] = acc[:n]

    return pl.pallas_call(
        body,
        out_shape=jax.ShapeDtypeStruct((n,), jnp.float32),
    )(partials_flat)


def kernel(long_edge_index, long_edge_length, atomic_charges):
    n = atomic_charges.shape[0]
    e = long_edge_length.shape[0]

    n_pad = ((n + 1 + 255) // 256) * 256
    e_pad = ((e + CH - 1) // CH) * CH

    length = long_edge_length.astype(jnp.float32)
    idx2 = long_edge_index.astype(jnp.int32)
    charges = atomic_charges.astype(jnp.float32)
    if e_pad != e:
        idx2 = jnp.pad(idx2, ((0, 0), (0, e_pad - e)), constant_values=n)
        length = jnp.pad(length, (0, e_pad - e), constant_values=1.0)
        charges = jnp.pad(charges, (0, n_pad - n))
    partials = _sc_coulomb(charges.shape[0], n_pad, e_pad)(
        idx2, length, charges)
    return _tc_sum(partials, n, n_pad)


# speedup vs baseline: 1.3036x; 1.0348x over previous
import functools

import jax
import jax.numpy as jnp
from jax import lax
from jax.experimental import pallas as pl
from jax.experimental.pallas import tpu as pltpu
from jax.experimental.pallas import tpu_sc as plsc

COULOMB_CONSTANT = 14.399645478425668
CUTOFF = 10.0
R_ON = 0.8 * CUTOFF
HALF_PI = 1.5707963267948966
PI = 3.141592653589793
SCALE = PI / (CUTOFF - R_ON)

C0 = 0.25 * COULOMB_CONSTANT
C3 = C0 * (-1.0 / 6.0)
C5 = C0 * (1.0 / 120.0)
C7 = C0 * (-1.0 / 5040.0)

NC = 2
NS = 16
NW = NC * NS
L = 16
CT = 512
CH = 5 * CT


def _sc_coulomb(n, n_pad, e_pad):
    total_ch = e_pad // CH
    ch_q, ch_r = divmod(total_ch, NW)

    mesh = plsc.VectorSubcoreMesh(core_axis_name="c", subcore_axis_name="s")

    @functools.partial(
        pl.kernel,
        out_type=jax.ShapeDtypeStruct((NW * n_pad,), jnp.float32),
        mesh=mesh,
        compiler_params=pltpu.CompilerParams(needs_layout_passes=False),
        scratch_types=[
            pltpu.VMEM((n_pad,), jnp.float32),
            pltpu.VMEM((n_pad,), jnp.float32),
            pltpu.VMEM((2, 2 * CH), jnp.int32),
            pltpu.VMEM((2 * CH,), jnp.float32),
            pltpu.VMEM_SHARED((n,), jnp.float32),
            pltpu.SemaphoreType.DMA,
            pltpu.SemaphoreType.DMA,
            pltpu.SemaphoreType.DMA,
        ],
    )
    def kern(idx2_hbm, length_hbm, charges_hbm, out_hbm,
             table, acc, ibuf, lbuf, shtbl, tsem, sem0, sem1):
        c = lax.axis_index("c")
        s = lax.axis_index("s")
        wid = c * NS + s
        nch_w = ch_q + jnp.where(wid < ch_r, 1, 0)
        start_w = wid * ch_q + jnp.minimum(wid, ch_r)

        def issue(j, half, sem):
            base = (start_w + j) * CH
            off = half * CH
            pltpu.async_copy(idx2_hbm.at[:, pl.ds(base, CH)],
                             ibuf.at[:, pl.ds(off, CH)], sem)
            pltpu.async_copy(length_hbm.at[pl.ds(base, CH)],
                             lbuf.at[pl.ds(off, CH)], sem)

        def drain(half, sem):
            off = half * CH
            pltpu.make_async_copy(idx2_hbm.at[:, pl.ds(0, CH)],
                                  ibuf.at[:, pl.ds(off, CH)], sem).wait()
            pltpu.make_async_copy(length_hbm.at[pl.ds(0, CH)],
                                  lbuf.at[pl.ds(off, CH)], sem).wait()

        def compute(half):
            off = half * CH

            @plsc.parallel_loop(0, CH, L, unroll=8)
            def _(i):
                cidx = ibuf[0, pl.ds(off + i, L)]
                nidx = ibuf[1, pl.ds(off + i, L)]
                d = lbuf[pl.ds(off + i, L)]
                q1 = plsc.load_gather(table, [cidx])
                q2 = plsc.load_gather(table, [nidx])
                x = jnp.clip((d - R_ON) * SCALE, 0.0, PI) - HALF_PI
                x2 = x * x
                sinx_c = x * (C0 + x2 * (C3 + x2 * (C5 + x2 * C7)))
                pair = (C0 - sinx_c) * q1 * q2 / d
                plsc.addupdate_scatter(acc, [cidx], pair)

        issue(0, 0, sem0)

        @pl.when(s == 0)
        def _():
            pltpu.sync_copy(charges_hbm, shtbl)

        zero16 = jnp.zeros((L,), jnp.float32)

        @plsc.parallel_loop(0, n_pad, L, unroll=8)
        def _(i):
            acc[pl.ds(i, L)] = zero16

        plsc.subcore_barrier()
        pltpu.async_copy(shtbl, table.at[pl.ds(0, n)], tsem).wait()

        @pl.loop(0, nch_w, step=2)
        def _(j):
            @pl.when(j + 1 < nch_w)
            def _():
                issue(j + 1, 1, sem1)

            drain(0, sem0)
            compute(0)

            @pl.when(j + 2 < nch_w)
            def _():
                issue(j + 2, 0, sem0)

            @pl.when(j + 1 < nch_w)
            def _():
                drain(1, sem1)
                compute(1)

        pltpu.sync_copy(acc, out_hbm.at[pl.ds(wid * n_pad, n_pad)])

    return kern


def _tc_sum(partials_flat, n, n_pad):
    def body(p_ref, o_ref):
        acc = p_ref[pl.ds(0, n_pad)]
        for w in range(1, NW):
            acc = acc + p_ref[pl.ds(w * n_pad, n_pad)]
        o_ref[...] = acc[:n]

    return pl.pallas_call(
        body,
        out_shape=jax.ShapeDtypeStruct((n,), jnp.float32),
    )(partials_flat)


def kernel(long_edge_index, long_edge_length, atomic_charges):
    n = atomic_charges.shape[0]
    e = long_edge_length.shape[0]

    n_pad = ((n + 1 + 255) // 256) * 256
    e_pad = ((e + CH - 1) // CH) * CH

    length = long_edge_length.astype(jnp.float32)
    idx2 = long_edge_index.astype(jnp.int32)
    charges = atomic_charges.astype(jnp.float32)
    if e_pad != e:
        idx2 = jnp.pad(idx2, ((0, 0), (0, e_pad - e)), constant_values=n)
        length = jnp.pad(length, (0, e_pad - e), constant_values=1.0)
        charges = jnp.pad(charges, (0, n_pad - n))
    partials = _sc_coulomb(charges.shape[0], n_pad, e_pad)(
        idx2, length, charges)
    return _tc_sum(partials, n, n_pad)
